# baseline (speedup 1.0000x reference)
import functools
import math

import jax
import jax.numpy as jnp
import numpy as np
from jax import lax
from jax.experimental import pallas as pl
from jax.experimental.pallas import tpu as pltpu
from jax.experimental.pallas import tpu_sc as plsc

D_MODEL = 2048
VOCAB = 50257
AST_VOCAB = 512
MAX_LEN = 2048
BATCH = 4
SEQ = 2048
N_TOK = BATCH * SEQ
SCALE = math.sqrt(D_MODEL)

NUM_CORES = 2
NUM_SUBCORES = 16
NUM_WORKERS = NUM_CORES * NUM_SUBCORES
PER_W = N_TOK // NUM_WORKERS
CHUNK = 8
N_CHUNKS = PER_W // CHUNK
LANES = 16
COL_ITERS = D_MODEL // LANES


def _pe_np() -> np.ndarray:
    position = np.arange(MAX_LEN)[:, None].astype(np.float32)
    div_term = np.exp(
        np.arange(0, D_MODEL, 2).astype(np.float32)
        * (-math.log(10000.0) / D_MODEL)
    )
    pe = np.zeros((MAX_LEN, D_MODEL), dtype=np.float32)
    pe[:, 0::2] = np.sin(position * div_term)
    pe[:, 1::2] = np.cos(position * div_term)
    return pe


_PE = _pe_np()


def _body(ids_hbm, aids_hbm, tok_hbm, ast_hbm, pe_hbm, out_hbm,
          idx_v, aidx_v, tok_buf, ast_buf, pe_buf, sem_t, sem_a):
    wid = lax.axis_index("s") * NUM_CORES + lax.axis_index("c")
    base = pl.multiple_of(wid * PER_W, PER_W)
    s0 = pl.multiple_of(lax.rem(base, SEQ), PER_W)

    pltpu.sync_copy(ids_hbm.at[pl.ds(base, PER_W)], idx_v)
    pltpu.sync_copy(aids_hbm.at[pl.ds(base, PER_W)], aidx_v)

    def chunk_step(j, carry):
        off = pl.multiple_of(j * CHUNK, CHUNK)
        cp_t = pltpu.make_async_copy(
            tok_hbm.at[idx_v.at[pl.ds(off, CHUNK)]], tok_buf, sem_t)
        cp_a = pltpu.make_async_copy(
            ast_hbm.at[aidx_v.at[pl.ds(off, CHUNK)]], ast_buf, sem_a)
        cp_t.start()
        cp_a.start()
        pltpu.sync_copy(pe_hbm.at[pl.ds(s0 + off, CHUNK)], pe_buf)
        cp_t.wait()
        cp_a.wait()

        def col_step(i, carry2):
            coff = pl.multiple_of(i * LANES, LANES)
            for r in range(CHUNK):
                v = (tok_buf[r, pl.ds(coff, LANES)]
                     + ast_buf[r, pl.ds(coff, LANES)]) * SCALE
                tok_buf[r, pl.ds(coff, LANES)] = v + pe_buf[r, pl.ds(coff, LANES)]
            return carry2

        lax.fori_loop(0, COL_ITERS, col_step, 0, unroll=2)
        pltpu.sync_copy(tok_buf, out_hbm.at[pl.ds(base + off, CHUNK)])
        return carry

    lax.fori_loop(0, N_CHUNKS, chunk_step, 0)


@jax.jit
def _encode(ids, aids, tok_table, ast_table, pe):
    mesh = plsc.VectorSubcoreMesh(
        core_axis_name="c", subcore_axis_name="s",
        num_cores=NUM_CORES, num_subcores=NUM_SUBCORES)
    f = pl.kernel(
        _body,
        out_type=jax.ShapeDtypeStruct((N_TOK, D_MODEL), jnp.float32),
        mesh=mesh,
        scratch_types=[
            pltpu.VMEM((PER_W,), jnp.int32),
            pltpu.VMEM((PER_W,), jnp.int32),
            pltpu.VMEM((CHUNK, D_MODEL), jnp.float32),
            pltpu.VMEM((CHUNK, D_MODEL), jnp.float32),
            pltpu.VMEM((CHUNK, D_MODEL), jnp.float32),
            pltpu.SemaphoreType.DMA,
            pltpu.SemaphoreType.DMA,
        ],
    )
    return f(ids, aids, tok_table, ast_table, pe)


def kernel(input_ids, ast_ids, token_embedding, ast_embedding):
    ids = input_ids.reshape(-1).astype(jnp.int32)
    aids = ast_ids.reshape(-1).astype(jnp.int32)
    pe = jnp.asarray(_PE)
    out = _encode(ids, aids, token_embedding, ast_embedding, pe)
    return out.reshape(BATCH, SEQ, D_MODEL)


# speedup vs baseline: 1.2336x; 1.2336x over previous
import math

import jax
import jax.numpy as jnp
import numpy as np
from jax import lax
from jax.experimental import pallas as pl
from jax.experimental.pallas import tpu as pltpu
from jax.experimental.pallas import tpu_sc as plsc

D_MODEL = 2048
VOCAB = 50257
AST_VOCAB = 512
MAX_LEN = 2048
BATCH = 4
SEQ = 2048
N_TOK = BATCH * SEQ
SCALE = math.sqrt(D_MODEL)

NUM_CORES = 2
NUM_SUBCORES = 16
NUM_WORKERS = NUM_CORES * NUM_SUBCORES
S_PER_W = SEQ // NUM_WORKERS
PER_W = N_TOK // NUM_WORKERS
CHUNK = 8
N_CHUNKS = PER_W // CHUNK
LANES = 16
COL_ITERS = D_MODEL // LANES


def _pe_np() -> np.ndarray:
    position = np.arange(MAX_LEN)[:, None].astype(np.float32)
    div_term = np.exp(
        np.arange(0, D_MODEL, 2).astype(np.float32)
        * (-math.log(10000.0) / D_MODEL)
    )
    pe = np.zeros((MAX_LEN, D_MODEL), dtype=np.float32)
    pe[:, 0::2] = np.sin(position * div_term)
    pe[:, 1::2] = np.cos(position * div_term)
    return pe


_PE = _pe_np()


def _body(ids_hbm, aids_hbm, tok_hbm, ast_hbm, pe_hbm, out_hbm,
          idx_v, aidx_v, tok_b, ast_b, out_b, pe_buf,
          sem_t0, sem_t1, sem_a0, sem_a1, sem_o0, sem_o1):
    sem_t = (sem_t0, sem_t1)
    sem_a = (sem_a0, sem_a1)
    sem_o = (sem_o0, sem_o1)

    wid = lax.axis_index("s") * NUM_CORES + lax.axis_index("c")
    wseq0 = pl.multiple_of(wid * S_PER_W, S_PER_W)

    for b in range(BATCH):
        pltpu.sync_copy(ids_hbm.at[pl.ds(b * SEQ + wseq0, S_PER_W)],
                        idx_v.at[pl.ds(b * S_PER_W, S_PER_W)])
        pltpu.sync_copy(aids_hbm.at[pl.ds(b * SEQ + wseq0, S_PER_W)],
                        aidx_v.at[pl.ds(b * S_PER_W, S_PER_W)])

    def gathers(t, p):
        b = lax.rem(t, BATCH)
        k = lax.div(t, BATCH)
        ioff = pl.multiple_of(b * S_PER_W + k * CHUNK, CHUNK)
        ct = pltpu.make_async_copy(
            tok_hbm.at[idx_v.at[pl.ds(ioff, CHUNK)]], tok_b.at[p], sem_t[p])
        ca = pltpu.make_async_copy(
            ast_hbm.at[aidx_v.at[pl.ds(ioff, CHUNK)]], ast_b.at[p], sem_a[p])
        return ct, ca

    def out_copy(t, p):
        b = lax.rem(t, BATCH)
        k = lax.div(t, BATCH)
        foff = pl.multiple_of(b * SEQ + wseq0 + k * CHUNK, CHUNK)
        return pltpu.make_async_copy(
            out_b.at[p], out_hbm.at[pl.ds(foff, CHUNK)], sem_o[p])

    pltpu.sync_copy(pe_hbm.at[pl.ds(wseq0, CHUNK)], pe_buf)
    ct0, ca0 = gathers(0, 0)
    ct0.start()
    ca0.start()

    def step(t2, carry):
        for p in range(2):
            t = 2 * t2 + p
            tn = t + 1

            @pl.when(tn < N_CHUNKS)
            def _():
                ct, ca = gathers(tn, 1 - p)
                ct.start()
                ca.start()

            @pl.when((t > 0) & (lax.rem(t, BATCH) == 0))
            def _():
                k = lax.div(t, BATCH)
                pltpu.sync_copy(
                    pe_hbm.at[pl.ds(wseq0 + k * CHUNK, CHUNK)], pe_buf)

            ct, ca = gathers(t, p)
            ct.wait()
            ca.wait()

            @pl.when(t >= 2)
            def _():
                out_copy(t - 2, p).wait()

            def col_step(i, carry2):
                coff = pl.multiple_of(i * LANES, LANES)
                for r in range(CHUNK):
                    v = (tok_b[p, r, pl.ds(coff, LANES)]
                         + ast_b[p, r, pl.ds(coff, LANES)]) * SCALE
                    out_b[p, r, pl.ds(coff, LANES)] = (
                        v + pe_buf[r, pl.ds(coff, LANES)])
                return carry2

            lax.fori_loop(0, COL_ITERS, col_step, 0, unroll=2)
            out_copy(t, p).start()
        return carry

    lax.fori_loop(0, N_CHUNKS // 2, step, 0)
    out_copy(N_CHUNKS - 2, 0).wait()
    out_copy(N_CHUNKS - 1, 1).wait()


@jax.jit
def _encode(ids, aids, tok_table, ast_table, pe):
    mesh = plsc.VectorSubcoreMesh(
        core_axis_name="c", subcore_axis_name="s",
        num_cores=NUM_CORES, num_subcores=NUM_SUBCORES)
    f = pl.kernel(
        _body,
        out_type=jax.ShapeDtypeStruct((N_TOK, D_MODEL), jnp.float32),
        mesh=mesh,
        scratch_types=[
            pltpu.VMEM((PER_W,), jnp.int32),
            pltpu.VMEM((PER_W,), jnp.int32),
            pltpu.VMEM((2, CHUNK, D_MODEL), jnp.float32),
            pltpu.VMEM((2, CHUNK, D_MODEL), jnp.float32),
            pltpu.VMEM((2, CHUNK, D_MODEL), jnp.float32),
            pltpu.VMEM((CHUNK, D_MODEL), jnp.float32),
            pltpu.SemaphoreType.DMA,
            pltpu.SemaphoreType.DMA,
            pltpu.SemaphoreType.DMA,
            pltpu.SemaphoreType.DMA,
            pltpu.SemaphoreType.DMA,
            pltpu.SemaphoreType.DMA,
        ],
    )
    return f(ids, aids, tok_table, ast_table, pe)


def kernel(input_ids, ast_ids, token_embedding, ast_embedding):
    ids = input_ids.reshape(-1).astype(jnp.int32)
    aids = ast_ids.reshape(-1).astype(jnp.int32)
    pe = jnp.asarray(_PE)
    out = _encode(ids, aids, token_embedding, ast_embedding, pe)
    return out.reshape(BATCH, SEQ, D_MODEL)


# speedup vs baseline: 2.0972x; 1.7001x over previous
import math

import jax
import jax.numpy as jnp
import numpy as np
from jax import lax
from jax.experimental import pallas as pl
from jax.experimental.pallas import tpu as pltpu
from jax.experimental.pallas import tpu_sc as plsc

D_MODEL = 2048
VOCAB = 50257
AST_VOCAB = 512
MAX_LEN = 2048
BATCH = 4
SEQ = 2048
N_TOK = BATCH * SEQ
SCALE = math.sqrt(D_MODEL)

NUM_CORES = 2
NUM_SUBCORES = 16
NUM_WORKERS = NUM_CORES * NUM_SUBCORES
S_PER_W = SEQ // NUM_WORKERS
PER_W = N_TOK // NUM_WORKERS
CHUNK = 8
N_CHUNKS = PER_W // CHUNK
LANES = 16
COL_ITERS = D_MODEL // LANES


def _pe_np() -> np.ndarray:
    position = np.arange(MAX_LEN)[:, None].astype(np.float32)
    div_term = np.exp(
        np.arange(0, D_MODEL, 2).astype(np.float32)
        * (-math.log(10000.0) / D_MODEL)
    )
    pe = np.zeros((MAX_LEN, D_MODEL), dtype=np.float32)
    pe[:, 0::2] = np.sin(position * div_term)
    pe[:, 1::2] = np.cos(position * div_term)
    return pe


_PE = _pe_np()


def _body(ids_hbm, aids_hbm, tok_hbm, ast_hbm, pe_hbm, out_hbm,
          idx_v, aidx_v, tok_b, ast_b, out_b, pe_buf,
          sem_t0, sem_t1, sem_a0, sem_a1, sem_o0, sem_o1):
    sem_t = (sem_t0, sem_t1)
    sem_a = (sem_a0, sem_a1)
    sem_o = (sem_o0, sem_o1)

    wid = lax.axis_index("s") * NUM_CORES + lax.axis_index("c")
    wseq0 = pl.multiple_of(wid * S_PER_W, S_PER_W)

    for b in range(BATCH):
        pltpu.sync_copy(ids_hbm.at[pl.ds(b * SEQ + wseq0, S_PER_W)],
                        idx_v.at[pl.ds(b * S_PER_W, S_PER_W)])
        pltpu.sync_copy(aids_hbm.at[pl.ds(b * SEQ + wseq0, S_PER_W)],
                        aidx_v.at[pl.ds(b * S_PER_W, S_PER_W)])

    def gathers(t, p):
        b = lax.rem(t, BATCH)
        k = lax.div(t, BATCH)
        ioff = pl.multiple_of(b * S_PER_W + k * CHUNK, CHUNK)
        ct = pltpu.make_async_copy(
            tok_hbm.at[idx_v.at[pl.ds(ioff, CHUNK)]], tok_b.at[p], sem_t[p])
        ca = pltpu.make_async_copy(
            ast_hbm.at[aidx_v.at[pl.ds(ioff, CHUNK)]], ast_b.at[p], sem_a[p])
        return ct, ca

    def out_copy(t, p):
        b = lax.rem(t, BATCH)
        k = lax.div(t, BATCH)
        foff = pl.multiple_of(b * SEQ + wseq0 + k * CHUNK, CHUNK)
        return pltpu.make_async_copy(
            out_b.at[p], out_hbm.at[pl.ds(foff, CHUNK)], sem_o[p])

    pltpu.sync_copy(pe_hbm.at[pl.ds(wseq0, CHUNK)], pe_buf)
    ct0, ca0 = gathers(0, 0)
    ct0.start()
    ca0.start()

    def step(t2, carry):
        for p in range(2):
            t = 2 * t2 + p
            tn = t + 1

            @pl.when(tn < N_CHUNKS)
            def _():
                ct, ca = gathers(tn, 1 - p)
                ct.start()
                ca.start()

            @pl.when((t > 0) & (lax.rem(t, BATCH) == 0))
            def _():
                k = lax.div(t, BATCH)
                pltpu.sync_copy(
                    pe_hbm.at[pl.ds(wseq0 + k * CHUNK, CHUNK)], pe_buf)

            ct, ca = gathers(t, p)
            ct.wait()
            ca.wait()

            @pl.when(t >= 2)
            def _():
                out_copy(t - 2, p).wait()

            @plsc.parallel_loop(0, COL_ITERS, 1, unroll=2)
            def _(i):
                coff = pl.multiple_of(i * LANES, LANES)
                for r in range(CHUNK):
                    v = (tok_b[p, r, pl.ds(coff, LANES)]
                         + ast_b[p, r, pl.ds(coff, LANES)]) * SCALE
                    out_b[p, r, pl.ds(coff, LANES)] = (
                        v + pe_buf[r, pl.ds(coff, LANES)])
            out_copy(t, p).start()
        return carry

    lax.fori_loop(0, N_CHUNKS // 2, step, 0)
    out_copy(N_CHUNKS - 2, 0).wait()
    out_copy(N_CHUNKS - 1, 1).wait()


@jax.jit
def _encode(ids, aids, tok_table, ast_table, pe):
    mesh = plsc.VectorSubcoreMesh(
        core_axis_name="c", subcore_axis_name="s",
        num_cores=NUM_CORES, num_subcores=NUM_SUBCORES)
    f = pl.kernel(
        _body,
        out_type=jax.ShapeDtypeStruct((N_TOK, D_MODEL), jnp.float32),
        mesh=mesh,
        scratch_types=[
            pltpu.VMEM((PER_W,), jnp.int32),
            pltpu.VMEM((PER_W,), jnp.int32),
            pltpu.VMEM((2, CHUNK, D_MODEL), jnp.float32),
            pltpu.VMEM((2, CHUNK, D_MODEL), jnp.float32),
            pltpu.VMEM((2, CHUNK, D_MODEL), jnp.float32),
            pltpu.VMEM((CHUNK, D_MODEL), jnp.float32),
            pltpu.SemaphoreType.DMA,
            pltpu.SemaphoreType.DMA,
            pltpu.SemaphoreType.DMA,
            pltpu.SemaphoreType.DMA,
            pltpu.SemaphoreType.DMA,
            pltpu.SemaphoreType.DMA,
        ],
    )
    return f(ids, aids, tok_table, ast_table, pe)


def kernel(input_ids, ast_ids, token_embedding, ast_embedding):
    ids = input_ids.reshape(-1).astype(jnp.int32)
    aids = ast_ids.reshape(-1).astype(jnp.int32)
    pe = jnp.asarray(_PE)
    out = _encode(ids, aids, token_embedding, ast_embedding, pe)
    return out.reshape(BATCH, SEQ, D_MODEL)
